# baseline (speedup 1.0000x reference)
import functools

import jax
import jax.numpy as jnp
from jax.experimental import pallas as pl
from jax.experimental.pallas import tpu as pltpu

RADIUS = 2.0
Y_ACC = 0.001
X_ACC = 0.001
NUM_DIMS = 8
TEMP = 50.0
EPS = 0.0001
BATCH = 2048
NY = int(2 * RADIUS / Y_ACC) + 1
NX = int(2 * RADIUS / X_ACC) + 1
NY_PAD = 4096
L2E = 1.4426950408889634
BLK_B = 512


def _fsm_kernel(x_ref, b_ref, out_ref):
    x = x_ref[...]
    xp = jnp.clip(x, -RADIUS + EPS, RADIUS - EPS)
    idx = jnp.round((xp + RADIUS) / (2.0 * RADIUS) * (NX - 1))
    xg = -RADIUS + idx * (2.0 * RADIUS / (NX - 1))

    y = (
        jax.lax.broadcasted_iota(jnp.int32, (1, NY_PAD), 1).astype(jnp.float32)
        * Y_ACC
        - RADIUS
    )
    bt = b_ref[...]

    tl2e = TEMP * L2E
    ltm = tl2e * (1.0 - jnp.min(bt, axis=-1, keepdims=True))

    ch = 1024
    nums, dens = [], []
    for k in range(NY_PAD // ch):
        yk = y[:, k * ch : (k + 1) * ch]
        btk = bt[:, k * ch : (k + 1) * ch]
        d = xg[:, :, None] - yk[None, :, :]
        s = jnp.exp2(jnp.square(d) * (-L2E)) - btk[None, :, :]
        e = jnp.exp2(s * tl2e - ltm[None, :, :])
        nums.append(jnp.sum(e * s, axis=-1))
        dens.append(jnp.sum(e, axis=-1))
    num = functools.reduce(jnp.add, nums)
    den = functools.reduce(jnp.add, dens)
    out_ref[...] = jnp.sum(num / den, axis=-1, keepdims=True)


@jax.jit
def kernel(X, theta):
    bt = jnp.concatenate(
        [
            jnp.zeros((NUM_DIMS, 1), jnp.float32),
            theta.T,
            jnp.full((NUM_DIMS, NY_PAD - NY), 1e30, jnp.float32),
        ],
        axis=1,
    )

    grid = BATCH // BLK_B
    out = pl.pallas_call(
        _fsm_kernel,
        grid=(grid,),
        in_specs=[
            pl.BlockSpec((BLK_B, NUM_DIMS), lambda i: (i, 0)),
            pl.BlockSpec((NUM_DIMS, NY_PAD), lambda i: (0, 0)),
        ],
        out_specs=pl.BlockSpec((BLK_B, 1), lambda i: (i, 0)),
        out_shape=jax.ShapeDtypeStruct((BATCH, 1), jnp.float32),
        compiler_params=pltpu.CompilerParams(
            dimension_semantics=("arbitrary",),
        ),
    )(X, bt)
    return out.reshape(BATCH)


# speedup vs baseline: 1.0930x; 1.0930x over previous
import functools

import jax
import jax.numpy as jnp
from jax.experimental import pallas as pl
from jax.experimental.pallas import tpu as pltpu

RADIUS = 2.0
Y_ACC = 0.001
X_ACC = 0.001
NUM_DIMS = 8
TEMP = 50.0
EPS = 0.0001
BATCH = 2048
NY = int(2 * RADIUS / Y_ACC) + 1
NX = int(2 * RADIUS / X_ACC) + 1
NY_PAD = 4096
L2E = 1.4426950408889634
BLK_B = 256


def _fsm_kernel(x_ref, b_ref, out_ref):
    x = x_ref[...]
    xp = jnp.clip(x, -RADIUS + EPS, RADIUS - EPS)
    idx = jnp.round((xp + RADIUS) / (2.0 * RADIUS) * (NX - 1))
    xg = -RADIUS + idx * (2.0 * RADIUS / (NX - 1))

    y = (
        jax.lax.broadcasted_iota(jnp.int32, (1, NY_PAD), 1).astype(jnp.float32)
        * Y_ACC
        - RADIUS
    )
    bt = b_ref[...]

    tl2e = TEMP * L2E
    ltm = tl2e * (1.0 - jnp.min(bt, axis=-1, keepdims=True))
    btt = bt * tl2e + ltm
    corr = jnp.sum(ltm) * (1.0 / tl2e)

    ch = 2048
    nums, dens = [], []
    for k in range(NY_PAD // ch):
        yk = y[:, k * ch : (k + 1) * ch]
        btk = btt[:, k * ch : (k + 1) * ch]
        d = xg[:, :, None] - yk[None, :, :]
        t = jnp.exp2(jnp.square(d) * (-L2E)) * tl2e - btk[None, :, :]
        e = jnp.exp2(t)
        nums.append(jnp.sum(e * t, axis=-1))
        dens.append(jnp.sum(e, axis=-1))
    num = functools.reduce(jnp.add, nums)
    den = functools.reduce(jnp.add, dens)
    out_ref[...] = (
        jnp.sum(num / den, axis=-1, keepdims=True) * (1.0 / tl2e) + corr
    )


@jax.jit
def kernel(X, theta):
    bt = jnp.concatenate(
        [
            jnp.zeros((NUM_DIMS, 1), jnp.float32),
            theta.T,
            jnp.full((NUM_DIMS, NY_PAD - NY), 1e30, jnp.float32),
        ],
        axis=1,
    )

    grid = BATCH // BLK_B
    out = pl.pallas_call(
        _fsm_kernel,
        grid=(grid,),
        in_specs=[
            pl.BlockSpec((BLK_B, NUM_DIMS), lambda i: (i, 0)),
            pl.BlockSpec((NUM_DIMS, NY_PAD), lambda i: (0, 0)),
        ],
        out_specs=pl.BlockSpec((BLK_B, 1), lambda i: (i, 0)),
        out_shape=jax.ShapeDtypeStruct((BATCH, 1), jnp.float32),
        compiler_params=pltpu.CompilerParams(
            dimension_semantics=("arbitrary",),
        ),
    )(X, bt)
    return out.reshape(BATCH)


# speedup vs baseline: 1.1575x; 1.0590x over previous
import functools

import jax
import jax.numpy as jnp
from jax.experimental import pallas as pl
from jax.experimental.pallas import tpu as pltpu

RADIUS = 2.0
Y_ACC = 0.001
X_ACC = 0.001
NUM_DIMS = 8
TEMP = 50.0
EPS = 0.0001
BATCH = 2048
NY = int(2 * RADIUS / Y_ACC) + 1
NX = int(2 * RADIUS / X_ACC) + 1
NY_PAD = 4096
L2E = 1.4426950408889634
BLK_B = 256


def _fsm_kernel(x_ref, b_ref, out_ref):
    x = x_ref[...]
    xp = jnp.clip(x, -RADIUS + EPS, RADIUS - EPS)
    idx = jnp.round((xp + RADIUS) / (2.0 * RADIUS) * (NX - 1))
    xg = -RADIUS + idx * (2.0 * RADIUS / (NX - 1))

    rl2e = 1.2011224087864498
    xgs = xg * rl2e
    y = (
        jax.lax.broadcasted_iota(jnp.int32, (1, NY_PAD), 1).astype(jnp.float32)
        * (Y_ACC * rl2e)
        - RADIUS * rl2e
    )
    bt = b_ref[...]

    tl2e = TEMP * L2E
    LT = 6.1726225627196225
    ltm = tl2e * (1.0 - jnp.min(bt, axis=-1, keepdims=True))
    btt = bt * tl2e + ltm
    corr = jnp.sum(ltm) * (1.0 / tl2e)

    ch = 2048
    nums, dens = [], []
    for k in range(NY_PAD // ch):
        yk = y[:, k * ch : (k + 1) * ch]
        btk = btt[:, k * ch : (k + 1) * ch]
        d = xgs[:, :, None] - yk[None, :, :]
        t = jnp.exp2(LT - jnp.square(d)) - btk[None, :, :]
        e = jnp.exp2(t)
        nums.append(jnp.sum(e * t, axis=-1))
        dens.append(jnp.sum(e, axis=-1))
    num = functools.reduce(jnp.add, nums)
    den = functools.reduce(jnp.add, dens)
    out_ref[...] = (
        jnp.sum(num / den, axis=-1, keepdims=True) * (1.0 / tl2e) + corr
    )


@jax.jit
def kernel(X, theta):
    bt = jnp.concatenate(
        [
            jnp.zeros((NUM_DIMS, 1), jnp.float32),
            theta.T,
            jnp.full((NUM_DIMS, NY_PAD - NY), 1e30, jnp.float32),
        ],
        axis=1,
    )

    grid = BATCH // BLK_B
    out = pl.pallas_call(
        _fsm_kernel,
        grid=(grid,),
        in_specs=[
            pl.BlockSpec((BLK_B, NUM_DIMS), lambda i: (i, 0)),
            pl.BlockSpec((NUM_DIMS, NY_PAD), lambda i: (0, 0)),
        ],
        out_specs=pl.BlockSpec((BLK_B, 1), lambda i: (i, 0)),
        out_shape=jax.ShapeDtypeStruct((BATCH, 1), jnp.float32),
        compiler_params=pltpu.CompilerParams(
            dimension_semantics=("arbitrary",),
        ),
    )(X, bt)
    return out.reshape(BATCH)
